# baseline (speedup 1.0000x reference)
import functools

import jax
import jax.numpy as jnp
from jax import lax
from jax.experimental import pallas as pl
from jax.experimental.pallas import tpu as pltpu
from jax.experimental.pallas import tpu_sc as plsc

WINDOW = 128
NUM_CORES = 2
NUM_SUBCORES = 16
NUM_WORKERS = NUM_CORES * NUM_SUBCORES

PACK_LANES = 4096


def _pack_body(wt_ref, out_ref, tmp_ref):
    tmp_ref[...] = jnp.swapaxes(wt_ref[...], 0, 1)
    for a in range(4):
        out_ref[:, 32 * a : 32 * (a + 1)] = tmp_ref[
            pl.Slice(a, PACK_LANES // 4, 4), :
        ]


def _unpack_body(x_ref, o_ref):
    x = x_ref[...]
    y = jnp.swapaxes(x, 0, 1)
    o_ref[...] = y.reshape(o_ref.shape)


def kernel(weights, indices):
    vocab, embed_dim = weights.shape
    batch, hist_len = indices.shape
    num_idx = batch * hist_len
    n_win = num_idx // WINDOW
    wpw = n_win // NUM_WORKERS
    ipw = wpw * WINDOW

    flat_idx = indices.reshape(num_idx)

    wt = weights.T
    n_pack = (vocab + PACK_LANES - 1) // PACK_LANES
    packed = pl.pallas_call(
        _pack_body,
        grid=(n_pack,),
        in_specs=[pl.BlockSpec((embed_dim, PACK_LANES), lambda i: (0, i))],
        out_specs=pl.BlockSpec((PACK_LANES // 4, 128), lambda i: (i, 0)),
        out_shape=jax.ShapeDtypeStruct((vocab // 4, 128), weights.dtype),
        scratch_shapes=[pltpu.VMEM((PACK_LANES, embed_dim), weights.dtype)],
    )(wt)
    w_lin = packed.reshape(vocab, embed_dim)

    mesh = plsc.VectorSubcoreMesh(core_axis_name="c", subcore_axis_name="s")

    @functools.partial(
        pl.kernel,
        mesh=mesh,
        compiler_params=pltpu.CompilerParams(use_tc_tiling_on_sc=False),
        out_type=jax.ShapeDtypeStruct((num_idx, embed_dim), weights.dtype),
        scratch_types=[
            pltpu.VMEM((ipw,), jnp.int32),
            pltpu.VMEM((WINDOW, embed_dim), jnp.float32),
            pltpu.SemaphoreType.DMA,
        ],
    )
    def gather_kernel(table_hbm, idx_hbm, out_hbm, idx_v, rows_v, sem):
        wid = lax.axis_index("s") * NUM_CORES + lax.axis_index("c")
        base = wid * ipw
        pltpu.sync_copy(idx_hbm.at[pl.ds(base, ipw)], idx_v)

        @pl.loop(0, wpw)
        def _(j):
            pltpu.async_copy(
                table_hbm.at[idx_v.at[pl.ds(j * WINDOW, WINDOW)]], rows_v, sem
            ).wait()
            pltpu.sync_copy(rows_v, out_hbm.at[pl.ds(base + j * WINDOW, WINDOW)])

    out = gather_kernel(w_lin, flat_idx)

    row = hist_len * embed_dim
    xb = out.reshape(batch, row)
    ot = pl.pallas_call(
        _unpack_body,
        grid=(batch // 128,),
        in_specs=[pl.BlockSpec((128, row), lambda i: (i, 0))],
        out_specs=pl.BlockSpec((hist_len, embed_dim, 128), lambda i: (0, 0, i)),
        out_shape=jax.ShapeDtypeStruct((hist_len, embed_dim, batch), weights.dtype),
    )(xb)
    return jnp.transpose(ot, (2, 0, 1))


# speedup vs baseline: 1.0636x; 1.0636x over previous
import functools

import jax
import jax.numpy as jnp
from jax import lax
from jax.experimental import pallas as pl
from jax.experimental.pallas import tpu as pltpu
from jax.experimental.pallas import tpu_sc as plsc

WINDOW = 128
NUM_CORES = 2
NUM_SUBCORES = 16
NUM_WORKERS = NUM_CORES * NUM_SUBCORES

PACK_LANES = 4096


def _pack_body(wt_ref, out_ref, tmp_ref):
    tmp_ref[...] = jnp.swapaxes(wt_ref[...], 0, 1)
    for a in range(4):
        out_ref[:, 32 * a : 32 * (a + 1)] = tmp_ref[
            pl.Slice(a, PACK_LANES // 4, 4), :
        ]


def _unpack_body(x_ref, o_ref):
    x = x_ref[...]
    y = jnp.swapaxes(x, 0, 1)
    o_ref[...] = y.reshape(o_ref.shape)


def kernel(weights, indices):
    vocab, embed_dim = weights.shape
    batch, hist_len = indices.shape
    num_idx = batch * hist_len
    n_win = num_idx // WINDOW
    wpw = n_win // NUM_WORKERS
    ipw = wpw * WINDOW

    flat_idx = indices.reshape(num_idx)

    wt = weights.T
    n_pack = (vocab + PACK_LANES - 1) // PACK_LANES
    packed = pl.pallas_call(
        _pack_body,
        grid=(n_pack,),
        in_specs=[pl.BlockSpec((embed_dim, PACK_LANES), lambda i: (0, i))],
        out_specs=pl.BlockSpec((PACK_LANES // 4, 128), lambda i: (i, 0)),
        out_shape=jax.ShapeDtypeStruct((vocab // 4, 128), weights.dtype),
        scratch_shapes=[pltpu.VMEM((PACK_LANES, embed_dim), weights.dtype)],
    )(wt)
    w_lin = packed.reshape(vocab, embed_dim)

    mesh = plsc.VectorSubcoreMesh(core_axis_name="c", subcore_axis_name="s")

    @functools.partial(
        pl.kernel,
        mesh=mesh,
        compiler_params=pltpu.CompilerParams(use_tc_tiling_on_sc=False),
        out_type=jax.ShapeDtypeStruct((num_idx, embed_dim), weights.dtype),
        scratch_types=[
            pltpu.VMEM((ipw,), jnp.int32),
            pltpu.VMEM((WINDOW, embed_dim), jnp.float32),
            pltpu.VMEM((WINDOW, embed_dim), jnp.float32),
            pltpu.SemaphoreType.DMA,
            pltpu.SemaphoreType.DMA,
            pltpu.SemaphoreType.DMA,
            pltpu.SemaphoreType.DMA,
        ],
    )
    def gather_kernel(
        table_hbm, idx_hbm, out_hbm, idx_v, rows_a, rows_b, ga_s, gb_s, oa_s, ob_s
    ):
        wid = lax.axis_index("s") * NUM_CORES + lax.axis_index("c")
        base = wid * ipw
        pltpu.sync_copy(idx_hbm.at[pl.ds(base, ipw)], idx_v)

        @pl.loop(0, wpw, step=2)
        def _(j):
            @pl.when(j > 0)
            def _():
                pltpu.make_async_copy(
                    rows_a, out_hbm.at[pl.ds(base, WINDOW)], oa_s
                ).wait()
                pltpu.make_async_copy(
                    rows_b, out_hbm.at[pl.ds(base, WINDOW)], ob_s
                ).wait()

            ga = pltpu.async_copy(
                table_hbm.at[idx_v.at[pl.ds(j * WINDOW, WINDOW)]], rows_a, ga_s
            )
            gb = pltpu.async_copy(
                table_hbm.at[idx_v.at[pl.ds((j + 1) * WINDOW, WINDOW)]],
                rows_b,
                gb_s,
            )
            ga.wait()
            pltpu.async_copy(
                rows_a, out_hbm.at[pl.ds(base + j * WINDOW, WINDOW)], oa_s
            )
            gb.wait()
            pltpu.async_copy(
                rows_b, out_hbm.at[pl.ds(base + (j + 1) * WINDOW, WINDOW)], ob_s
            )

        pltpu.make_async_copy(rows_a, out_hbm.at[pl.ds(base, WINDOW)], oa_s).wait()
        pltpu.make_async_copy(rows_b, out_hbm.at[pl.ds(base, WINDOW)], ob_s).wait()

    out = gather_kernel(w_lin, flat_idx)

    row = hist_len * embed_dim
    xb = out.reshape(batch, row)
    ot = pl.pallas_call(
        _unpack_body,
        grid=(batch // 128,),
        in_specs=[pl.BlockSpec((128, row), lambda i: (i, 0))],
        out_specs=pl.BlockSpec((hist_len, embed_dim, 128), lambda i: (0, 0, i)),
        out_shape=jax.ShapeDtypeStruct((hist_len, embed_dim, batch), weights.dtype),
    )(xb)
    return jnp.transpose(ot, (2, 0, 1))


# speedup vs baseline: 1.1246x; 1.0574x over previous
import functools

import jax
import jax.numpy as jnp
from jax import lax
from jax.experimental import pallas as pl
from jax.experimental.pallas import tpu as pltpu
from jax.experimental.pallas import tpu_sc as plsc

WINDOW = 128
NUM_CORES = 2
NUM_SUBCORES = 16
NUM_WORKERS = NUM_CORES * NUM_SUBCORES

PACK_LANES = 8192


def _pack_body(wt_ref, out_ref, tmp_ref):
    tmp_ref[...] = jnp.swapaxes(wt_ref[...], 0, 1)
    for a in range(4):
        out_ref[:, 32 * a : 32 * (a + 1)] = tmp_ref[
            pl.Slice(a, PACK_LANES // 4, 4), :
        ]


def _unpack_body(x_ref, o_ref):
    x = x_ref[...]
    y = jnp.swapaxes(x, 0, 1)
    o_ref[...] = y.reshape(o_ref.shape)


def kernel(weights, indices):
    vocab, embed_dim = weights.shape
    batch, hist_len = indices.shape
    num_idx = batch * hist_len
    n_win = num_idx // WINDOW
    wpw = n_win // NUM_WORKERS
    ipw = wpw * WINDOW

    flat_idx = indices.reshape(num_idx)

    wt = weights.T
    n_pack = (vocab + PACK_LANES - 1) // PACK_LANES
    packed = pl.pallas_call(
        _pack_body,
        grid=(n_pack,),
        in_specs=[pl.BlockSpec((embed_dim, PACK_LANES), lambda i: (0, i))],
        out_specs=pl.BlockSpec((PACK_LANES // 4, 128), lambda i: (i, 0)),
        out_shape=jax.ShapeDtypeStruct((vocab // 4, 128), weights.dtype),
        scratch_shapes=[pltpu.VMEM((PACK_LANES, embed_dim), weights.dtype)],
    )(wt)
    w_lin = packed.reshape(vocab, embed_dim)

    mesh = plsc.VectorSubcoreMesh(core_axis_name="c", subcore_axis_name="s")

    @functools.partial(
        pl.kernel,
        mesh=mesh,
        compiler_params=pltpu.CompilerParams(use_tc_tiling_on_sc=False),
        out_type=jax.ShapeDtypeStruct((num_idx, embed_dim), weights.dtype),
        scratch_types=[
            pltpu.VMEM((ipw,), jnp.int32),
            pltpu.VMEM((WINDOW, embed_dim), jnp.float32),
            pltpu.VMEM((WINDOW, embed_dim), jnp.float32),
            pltpu.SemaphoreType.DMA,
            pltpu.SemaphoreType.DMA,
            pltpu.SemaphoreType.DMA,
            pltpu.SemaphoreType.DMA,
        ],
    )
    def gather_kernel(
        table_hbm, idx_hbm, out_hbm, idx_v, rows_a, rows_b, ga_s, gb_s, oa_s, ob_s
    ):
        wid = lax.axis_index("s") * NUM_CORES + lax.axis_index("c")
        base = wid * ipw
        pltpu.sync_copy(idx_hbm.at[pl.ds(base, ipw)], idx_v)

        @pl.loop(0, wpw, step=2)
        def _(j):
            @pl.when(j > 0)
            def _():
                pltpu.make_async_copy(
                    rows_a, out_hbm.at[pl.ds(base, WINDOW)], oa_s
                ).wait()
                pltpu.make_async_copy(
                    rows_b, out_hbm.at[pl.ds(base, WINDOW)], ob_s
                ).wait()

            ga = pltpu.async_copy(
                table_hbm.at[idx_v.at[pl.ds(j * WINDOW, WINDOW)]], rows_a, ga_s
            )
            gb = pltpu.async_copy(
                table_hbm.at[idx_v.at[pl.ds((j + 1) * WINDOW, WINDOW)]],
                rows_b,
                gb_s,
            )
            ga.wait()
            pltpu.async_copy(
                rows_a, out_hbm.at[pl.ds(base + j * WINDOW, WINDOW)], oa_s
            )
            gb.wait()
            pltpu.async_copy(
                rows_b, out_hbm.at[pl.ds(base + (j + 1) * WINDOW, WINDOW)], ob_s
            )

        pltpu.make_async_copy(rows_a, out_hbm.at[pl.ds(base, WINDOW)], oa_s).wait()
        pltpu.make_async_copy(rows_b, out_hbm.at[pl.ds(base, WINDOW)], ob_s).wait()

    out = gather_kernel(w_lin, flat_idx)

    row = hist_len * embed_dim
    xb = out.reshape(batch, row)
    ot = pl.pallas_call(
        _unpack_body,
        grid=(batch // 128,),
        in_specs=[pl.BlockSpec((128, row), lambda i: (i, 0))],
        out_specs=pl.BlockSpec((hist_len, embed_dim, 128), lambda i: (0, 0, i)),
        out_shape=jax.ShapeDtypeStruct((hist_len, embed_dim, batch), weights.dtype),
    )(xb)
    return jnp.transpose(ot, (2, 0, 1))


# speedup vs baseline: 1.1336x; 1.0079x over previous
import functools

import jax
import jax.numpy as jnp
from jax import lax
from jax.experimental import pallas as pl
from jax.experimental.pallas import tpu as pltpu
from jax.experimental.pallas import tpu_sc as plsc

WINDOW = 128
NUM_CORES = 2
NUM_SUBCORES = 16
NUM_WORKERS = NUM_CORES * NUM_SUBCORES

PACK_LANES = 16384


def _pack_body(wt_ref, out_ref, tmp_ref):
    tmp_ref[...] = jnp.swapaxes(wt_ref[...], 0, 1)
    for a in range(4):
        out_ref[:, 32 * a : 32 * (a + 1)] = tmp_ref[
            pl.Slice(a, PACK_LANES // 4, 4), :
        ]


def _unpack_body(x_ref, o_ref):
    x = x_ref[...]
    y = jnp.swapaxes(x, 0, 1)
    o_ref[...] = y.reshape(o_ref.shape)


def kernel(weights, indices):
    vocab, embed_dim = weights.shape
    batch, hist_len = indices.shape
    num_idx = batch * hist_len
    n_win = num_idx // WINDOW
    wpw = n_win // NUM_WORKERS
    ipw = wpw * WINDOW

    flat_idx = indices.reshape(num_idx)

    wt = weights.T
    n_pack = (vocab + PACK_LANES - 1) // PACK_LANES
    packed = pl.pallas_call(
        _pack_body,
        grid=(n_pack,),
        in_specs=[pl.BlockSpec((embed_dim, PACK_LANES), lambda i: (0, i))],
        out_specs=pl.BlockSpec((PACK_LANES // 4, 128), lambda i: (i, 0)),
        out_shape=jax.ShapeDtypeStruct((vocab // 4, 128), weights.dtype),
        scratch_shapes=[pltpu.VMEM((PACK_LANES, embed_dim), weights.dtype)],
    )(wt)
    w_lin = packed.reshape(vocab, embed_dim)

    mesh = plsc.VectorSubcoreMesh(core_axis_name="c", subcore_axis_name="s")

    @functools.partial(
        pl.kernel,
        mesh=mesh,
        compiler_params=pltpu.CompilerParams(use_tc_tiling_on_sc=False),
        out_type=jax.ShapeDtypeStruct((num_idx, embed_dim), weights.dtype),
        scratch_types=[
            pltpu.VMEM((ipw,), jnp.int32),
            pltpu.VMEM((WINDOW, embed_dim), jnp.float32),
            pltpu.VMEM((WINDOW, embed_dim), jnp.float32),
            pltpu.SemaphoreType.DMA,
            pltpu.SemaphoreType.DMA,
            pltpu.SemaphoreType.DMA,
            pltpu.SemaphoreType.DMA,
        ],
    )
    def gather_kernel(
        table_hbm, idx_hbm, out_hbm, idx_v, rows_a, rows_b, ga_s, gb_s, oa_s, ob_s
    ):
        wid = lax.axis_index("s") * NUM_CORES + lax.axis_index("c")
        base = wid * ipw
        pltpu.sync_copy(idx_hbm.at[pl.ds(base, ipw)], idx_v)

        @pl.loop(0, wpw, step=2)
        def _(j):
            @pl.when(j > 0)
            def _():
                pltpu.make_async_copy(
                    rows_a, out_hbm.at[pl.ds(base, WINDOW)], oa_s
                ).wait()
                pltpu.make_async_copy(
                    rows_b, out_hbm.at[pl.ds(base, WINDOW)], ob_s
                ).wait()

            ga = pltpu.async_copy(
                table_hbm.at[idx_v.at[pl.ds(j * WINDOW, WINDOW)]], rows_a, ga_s
            )
            gb = pltpu.async_copy(
                table_hbm.at[idx_v.at[pl.ds((j + 1) * WINDOW, WINDOW)]],
                rows_b,
                gb_s,
            )
            ga.wait()
            pltpu.async_copy(
                rows_a, out_hbm.at[pl.ds(base + j * WINDOW, WINDOW)], oa_s
            )
            gb.wait()
            pltpu.async_copy(
                rows_b, out_hbm.at[pl.ds(base + (j + 1) * WINDOW, WINDOW)], ob_s
            )

        pltpu.make_async_copy(rows_a, out_hbm.at[pl.ds(base, WINDOW)], oa_s).wait()
        pltpu.make_async_copy(rows_b, out_hbm.at[pl.ds(base, WINDOW)], ob_s).wait()

    out = gather_kernel(w_lin, flat_idx)

    row = hist_len * embed_dim
    xb = out.reshape(batch, row)
    ot = pl.pallas_call(
        _unpack_body,
        grid=(batch // 128,),
        in_specs=[pl.BlockSpec((128, row), lambda i: (i, 0))],
        out_specs=pl.BlockSpec((hist_len, embed_dim, 128), lambda i: (0, 0, i)),
        out_shape=jax.ShapeDtypeStruct((hist_len, embed_dim, batch), weights.dtype),
    )(xb)
    return jnp.transpose(ot, (2, 0, 1))
